# baseline (speedup 1.0000x reference)
import jax
import jax.numpy as jnp
from jax.experimental import pallas as pl
from jax.experimental.pallas import tpu as pltpu

EPS = 1e-08

_B, _C, _H, _W = 32, 64, 256, 256
_HW = _H * _W
_N = _B * _HW

_SGI = _B

_NL = 32768
_NGJ = _HW // _NL


def _stats_body(x_ref, sum_ref, sq_ref):
    xb = x_ref[...]
    sum_ref[...] = jnp.sum(xb, axis=1, keepdims=True)[None]
    sq_ref[...] = jnp.sum(xb * xb, axis=1, keepdims=True)[None]


def _norm_body(ps_ref, pq_ref, x_ref, o_ref):
    total = jnp.sum(ps_ref[...], axis=0)
    totsq = jnp.sum(pq_ref[...], axis=0)
    mean = total / _N
    m2 = totsq - total * mean
    var = jnp.maximum(m2 / (_N - 1), EPS)
    rstd = jax.lax.rsqrt(var + EPS)
    o_ref[...] = (x_ref[...] - mean) * rstd


def kernel(x):
    x2 = x.reshape(_B * _C, _HW)

    ps, pq = pl.pallas_call(
        _stats_body,
        grid=(_SGI,),
        in_specs=[pl.BlockSpec((_C, _HW), lambda i: (i, 0))],
        out_specs=[
            pl.BlockSpec((1, _C, 1), lambda i: (i, 0, 0)),
            pl.BlockSpec((1, _C, 1), lambda i: (i, 0, 0)),
        ],
        out_shape=[
            jax.ShapeDtypeStruct((_SGI, _C, 1), jnp.float32),
            jax.ShapeDtypeStruct((_SGI, _C, 1), jnp.float32),
        ],
        compiler_params=pltpu.CompilerParams(
            dimension_semantics=("parallel",),
            vmem_limit_bytes=48 * 1024 * 1024,
        ),
        name="welford_stats",
    )(x2)

    z2 = pl.pallas_call(
        _norm_body,
        grid=(_B, _NGJ),
        in_specs=[
            pl.BlockSpec((_SGI, _C, 1), lambda i, j: (0, 0, 0)),
            pl.BlockSpec((_SGI, _C, 1), lambda i, j: (0, 0, 0)),
            pl.BlockSpec((_C, _NL), lambda i, j: (i, j)),
        ],
        out_specs=pl.BlockSpec((_C, _NL), lambda i, j: (i, j)),
        out_shape=jax.ShapeDtypeStruct((_B * _C, _HW), jnp.float32),
        compiler_params=pltpu.CompilerParams(
            dimension_semantics=("parallel", "arbitrary"),
            vmem_limit_bytes=48 * 1024 * 1024,
        ),
        name="welford_normalize",
    )(ps, pq, x2)

    return z2.reshape(x.shape)


# speedup vs baseline: 2.5561x; 2.5561x over previous
import jax
import jax.numpy as jnp
from jax.experimental import pallas as pl
from jax.experimental.pallas import tpu as pltpu

EPS = 1e-08

_B, _C, _H, _W = 32, 64, 256, 256
_N = _B * _H * _W

_NH = 128
_NGJ = _H // _NH


def _stats_body(x_ref, sum_ref, sq_ref):
    xb = x_ref[...]
    sum_ref[...] = jnp.sum(xb, axis=(1, 2)).reshape(1, _C, 1)
    sq_ref[...] = jnp.sum(xb * xb, axis=(1, 2)).reshape(1, _C, 1)


def _norm_body(ps_ref, pq_ref, x_ref, o_ref):
    total = jnp.sum(ps_ref[...], axis=0)
    totsq = jnp.sum(pq_ref[...], axis=0)
    mean = total / _N
    m2 = totsq - total * mean
    var = jnp.maximum(m2 / (_N - 1), EPS)
    rstd = jax.lax.rsqrt(var + EPS)
    o_ref[...] = (x_ref[...] - mean[:, :, None]) * rstd[:, :, None]


def kernel(x):
    x3 = x.reshape(_B * _C, _H, _W)

    ps, pq = pl.pallas_call(
        _stats_body,
        grid=(_B,),
        in_specs=[pl.BlockSpec((_C, _H, _W), lambda i: (i, 0, 0))],
        out_specs=[
            pl.BlockSpec((1, _C, 1), lambda i: (i, 0, 0)),
            pl.BlockSpec((1, _C, 1), lambda i: (i, 0, 0)),
        ],
        out_shape=[
            jax.ShapeDtypeStruct((_B, _C, 1), jnp.float32),
            jax.ShapeDtypeStruct((_B, _C, 1), jnp.float32),
        ],
        compiler_params=pltpu.CompilerParams(
            dimension_semantics=("parallel",),
            vmem_limit_bytes=48 * 1024 * 1024,
        ),
        name="welford_stats",
    )(x3)

    z3 = pl.pallas_call(
        _norm_body,
        grid=(_B, _NGJ),
        in_specs=[
            pl.BlockSpec((_B, _C, 1), lambda i, j: (0, 0, 0)),
            pl.BlockSpec((_B, _C, 1), lambda i, j: (0, 0, 0)),
            pl.BlockSpec((_C, _NH, _W), lambda i, j: (i, j, 0)),
        ],
        out_specs=pl.BlockSpec((_C, _NH, _W), lambda i, j: (i, j, 0)),
        out_shape=jax.ShapeDtypeStruct((_B * _C, _H, _W), jnp.float32),
        compiler_params=pltpu.CompilerParams(
            dimension_semantics=("parallel", "arbitrary"),
            vmem_limit_bytes=48 * 1024 * 1024,
        ),
        name="welford_normalize",
    )(ps, pq, x3)

    return z3.reshape(x.shape)
